# Initial kernel scaffold; baseline (speedup 1.0000x reference)
import jax
import jax.numpy as jnp
from jax.experimental import pallas as pl


def kernel(input, others):
    raise NotImplementedError("write your pallas kernel here")


# speedup vs baseline: 1.5483x; 1.5483x over previous
import functools

import jax
import jax.numpy as jnp
from jax import lax
from jax.experimental import pallas as pl
from jax.experimental.pallas import tpu as pltpu
from jax.experimental.pallas import tpu_sc as plsc

V = 1_000_000
D = 32
B = 16384 * 26
NC = 2
NS = 16
NW = NC * NS
CHUNK = 128
GRP = 8
GROUP = GRP * CHUNK
B_PER_W = B // NW
NG = B_PER_W // GROUP

assert B_PER_W * NW == B and NG * GROUP == B_PER_W


@functools.partial(
    pl.kernel,
    mesh=plsc.VectorSubcoreMesh(core_axis_name="c", subcore_axis_name="s"),
    out_type=jax.ShapeDtypeStruct((B, D), jnp.float32),
    scratch_types=[
        pltpu.VMEM((GRP, CHUNK), jnp.int32),
        pltpu.VMEM((GROUP, D), jnp.float32),
        pltpu.SemaphoreType.DMA,
    ],
    compiler_params=pltpu.CompilerParams(use_tc_tiling_on_sc=False),
)
def _gather_kernel(idx_hbm, table_hbm, out_hbm, idx_v, rows_v, sem):
    wid = lax.axis_index("s") * NC + lax.axis_index("c")
    row0 = wid * (B_PER_W // CHUNK)
    base = wid * B_PER_W

    def group_body(g, carry):
        pltpu.sync_copy(idx_hbm.at[pl.ds(row0 + g * GRP, GRP)], idx_v)
        handles = [
            pltpu.async_copy(
                table_hbm.at[idx_v.at[j]],
                rows_v.at[pl.ds(j * CHUNK, CHUNK)],
                sem,
            )
            for j in range(GRP)
        ]
        for h in handles:
            h.wait()
        pltpu.sync_copy(rows_v, out_hbm.at[pl.ds(base + g * GROUP, GROUP)])
        return carry

    lax.fori_loop(0, NG, group_body, 0)


def kernel(input, others):
    idx = input.astype(jnp.int32).reshape(B // CHUNK, CHUNK)
    out = _gather_kernel(idx, others)
    return out.reshape(input.shape[0], input.shape[1], D)
